# baseline (speedup 1.0000x reference)
import jax
import jax.numpy as jnp
from jax.experimental import pallas as pl

_ADV_W = 0.6
_NORM_W = 0.05
_DATA_W = 0.25


def _loss_kernel(d_ref, fake_ref, real_ref, out_ref):
    n_slice = 4 * 2048 * 3
    adv = -jnp.sum(jnp.log(d_ref[...] + 1e-08)) / d_ref.size
    diff = real_ref[...] - fake_ref[...]
    row = jax.lax.broadcasted_iota(jnp.int32, diff.shape, 0)
    lane = jax.lax.broadcasted_iota(jnp.int32, diff.shape, 1)
    ch = (row * 128 + lane) % 6
    w_sqrt = jnp.where(ch < 3, (_DATA_W / n_slice) ** 0.5,
                       (_NORM_W / n_slice) ** 0.5)
    t = diff * w_sqrt
    out_ref[...] = jnp.reshape(_ADV_W * adv + jnp.sum(t * t), (1, 1))


def kernel(D_output_fake, fake_data, real_data):
    fake_flat = jnp.reshape(fake_data, (384, 128))
    real_flat = jnp.reshape(real_data, (384, 128))
    out = pl.pallas_call(
        _loss_kernel,
        out_shape=jax.ShapeDtypeStruct((1, 1), jnp.float32),
    )(D_output_fake, fake_flat, real_flat)
    return out[0, 0]


# speedup vs baseline: 2.6733x; 2.6733x over previous
import jax
import jax.numpy as jnp
from jax.experimental import pallas as pl

_ADV_W = 0.6
_NORM_W = 0.05
_DATA_W = 0.25


def _loss_kernel(d_ref, fake_ref, real_ref, out_ref):
    n_slice = 4 * 2048 * 3
    adv = -jnp.sum(jnp.log(d_ref[...] + 1e-08)) / d_ref.size
    diff = real_ref[...] - fake_ref[...]
    ch = jax.lax.broadcasted_iota(jnp.int32, diff.shape, 1)
    w_sqrt = jnp.where(ch < 3, (_DATA_W / n_slice) ** 0.5,
                       (_NORM_W / n_slice) ** 0.5)
    t = diff * w_sqrt
    out_ref[...] = jnp.reshape(_ADV_W * adv + jnp.sum(t * t), (1, 1))


def kernel(D_output_fake, fake_data, real_data):
    fake_t = jnp.transpose(fake_data, (0, 2, 1))
    real_t = jnp.transpose(real_data, (0, 2, 1))
    out = pl.pallas_call(
        _loss_kernel,
        out_shape=jax.ShapeDtypeStruct((1, 1), jnp.float32),
    )(D_output_fake, fake_t, real_t)
    return out[0, 0]
